# baseline (speedup 1.0000x reference)
import functools

import jax
import jax.numpy as jnp
from jax import lax
from jax.experimental import pallas as pl
from jax.experimental.pallas import tpu as pltpu
from jax.experimental.pallas import tpu_sc as plsc

_DEPTH = 7
_RES_SKIPS = (3, 6)
_WIDTH = 128
_GW = 64
_GLAYERS = 4
_K = 16
_LANES = 16
_SCALE = 0.01


def _silu(x):
    return x * jax.nn.sigmoid(x)


def _pack_half(x):
    b = x.astype(jnp.bfloat16)
    lo = lax.bitcast_convert_type(b[:, :_GW // 2], jnp.uint16).astype(jnp.uint32)
    hi = lax.bitcast_convert_type(b[:, _GW // 2:], jnp.uint16).astype(jnp.uint32)
    return lax.bitcast_convert_type(lo | (hi << 16), jnp.int32)


def _unpack_half(w):
    wu = lax.bitcast_convert_type(w, jnp.uint32)
    lo = lax.bitcast_convert_type((wu & 0xFFFF).astype(jnp.uint16),
                                  jnp.bfloat16)
    hi = lax.bitcast_convert_type((wu >> 16).astype(jnp.uint16), jnp.bfloat16)
    return lo, hi


def _dot(a, b):
    return lax.dot_general(a, b, (((1,), (0,)), ((), ())),
                           preferred_element_type=jnp.float32)


def _full_spec(shape):
    zeros = (0,) * len(shape)
    return pl.BlockSpec(shape, lambda i, _z=zeros: _z)


def _row_spec(blk, cols):
    return pl.BlockSpec((blk, cols), lambda i: (i, 0))


def _dot_t(at, b):
    return lax.dot_general(at, b, (((0,), (0,)), ((), ())),
                           preferred_element_type=jnp.float32)


def _trunk_body(coordsT_ref, condT_ref, BB_ref, W0c_ref, W0s_ref, W0k_ref,
                b0_ref, Wt_ref, bt_ref, Wg_ref, bg_ref, Wb_ref, bb_ref,
                Wgp_ref, bgp_ref, h_ref, hg_ref):
    xT = coordsT_ref[...]
    condT = condT_ref[...].astype(jnp.bfloat16)
    PT = _dot_t(BB_ref[...], xT)
    S = jnp.sin(PT)
    C = jnp.cos(PT)
    h = (_dot_t(xT, W0c_ref[...]) + _dot_t(S, W0s_ref[...])
         + _dot_t(C, W0k_ref[...]) + b0_ref[...])

    def film(v, l):
        gamma = _dot_t(condT, Wg_ref[l]) + bg_ref[l]
        beta = _dot_t(condT, Wb_ref[l]) + bb_ref[l]
        return v * (1.0 + gamma) + beta

    h = _silu(film(h, 0))
    h_skip = h
    for l in range(_DEPTH - 1):
        h = _silu(film(_dot(h.astype(jnp.bfloat16), Wt_ref[l]) + bt_ref[l],
                       l + 1))
        if (l + 1) in _RES_SKIPS:
            h = h + h_skip
    h_ref[...] = h
    hg_ref[...] = _pack_half(_silu(_dot(h.astype(jnp.bfloat16), Wgp_ref[...])
                                   + bgp_ref[...]))


def _col_spec(rows, blk):
    return pl.BlockSpec((rows, blk), lambda i: (0, i))


def _trunk(coordsT, condT, BB, W0c, W0s, W0k, b0, Wt, bt, Wg, bg, Wb, bb,
           Wgp, bgp, blk):
    n = coordsT.shape[1]
    grid = n // blk
    args = (coordsT, condT, BB, W0c, W0s, W0k, b0, Wt, bt, Wg, bg, Wb, bb,
            Wgp, bgp)
    in_specs = [_col_spec(3, blk), _col_spec(condT.shape[0], blk)]
    in_specs += [_full_spec(a.shape) for a in args[2:]]
    return pl.pallas_call(
        _trunk_body,
        grid=(grid,),
        in_specs=in_specs,
        out_specs=[_row_spec(blk, _WIDTH), _row_spec(blk, _GW // 2)],
        out_shape=[jax.ShapeDtypeStruct((n, _WIDTH), jnp.float32),
                   jax.ShapeDtypeStruct((n, _GW // 2), jnp.int32)],
    )(*args)


def _gc_body(agg_ref, hgp_ref, Wnlo_ref, Wnhi_ref, Wslo_ref, Wshi_ref, b_ref,
             out_ref):
    agg = agg_ref[...].astype(jnp.bfloat16)
    alo = agg[:, :_GW // 2]
    ahi = agg[:, _GW // 2:]
    lo, hi = _unpack_half(hgp_ref[...])
    out_ref[...] = _pack_half(_silu(_dot(alo, Wnlo_ref[...])
                                    + _dot(ahi, Wnhi_ref[...])
                                    + _dot(lo, Wslo_ref[...])
                                    + _dot(hi, Wshi_ref[...]) + b_ref[...]))


def _gc(agg, hgp, Wnlo_l, Wnhi_l, Wslo_l, Wshi_l, b_l, blk):
    n = hgp.shape[0]
    grid = n // blk
    return pl.pallas_call(
        _gc_body,
        grid=(grid,),
        in_specs=[_row_spec(blk, _GW), _row_spec(blk, _GW // 2),
                  _full_spec(Wnlo_l.shape), _full_spec(Wnhi_l.shape),
                  _full_spec(Wslo_l.shape), _full_spec(Wshi_l.shape),
                  _full_spec(b_l.shape)],
        out_specs=_row_spec(blk, _GW // 2),
        out_shape=jax.ShapeDtypeStruct((n, _GW // 2), jnp.int32),
    )(agg, hgp, Wnlo_l, Wnhi_l, Wslo_l, Wshi_l, b_l)


def _final_body(h_ref, agg_ref, hgp_ref, Wnlo_ref, Wnhi_ref, Wslo_ref,
                Wshi_ref, b_ref, Wout_ref, Wbo_ref, bout_ref, u_ref):
    agg = agg_ref[...].astype(jnp.bfloat16)
    alo = agg[:, :_GW // 2]
    ahi = agg[:, _GW // 2:]
    lo, hi = _unpack_half(hgp_ref[...])
    hg4 = _silu(_dot(alo, Wnlo_ref[...]) + _dot(ahi, Wnhi_ref[...])
                + _dot(lo, Wslo_ref[...]) + _dot(hi, Wshi_ref[...])
                + b_ref[...])
    contract_last = (((0,), (1,)), ((), ()))
    uT = (lax.dot_general(Wout_ref[...], h_ref[...], contract_last,
                          preferred_element_type=jnp.float32)
          + lax.dot_general(Wbo_ref[...], hg4, contract_last,
                            preferred_element_type=jnp.float32)
          + bout_ref[...]) * _SCALE
    u_ref[...] = uT


def _final(h, agg, hgp, Wnlo_l, Wnhi_l, Wslo_l, Wshi_l, b_l, Wout, Wbo,
           bout2, blk):
    n = h.shape[0]
    grid = n // blk
    out_dim = Wout.shape[1]
    return pl.pallas_call(
        _final_body,
        grid=(grid,),
        in_specs=[_row_spec(blk, _WIDTH), _row_spec(blk, _GW),
                  _row_spec(blk, _GW // 2), _full_spec(Wnlo_l.shape),
                  _full_spec(Wnhi_l.shape),
                  _full_spec(Wslo_l.shape), _full_spec(Wshi_l.shape),
                  _full_spec(b_l.shape),
                  _full_spec(Wout.shape), _full_spec(Wbo.shape),
                  _full_spec(bout2.shape)],
        out_specs=_col_spec(out_dim, blk),
        out_shape=jax.ShapeDtypeStruct((out_dim, n), jnp.float32),
    )(h, agg, hgp, Wnlo_l, Wnhi_l, Wslo_l, Wshi_l, b_l, Wout, Wbo, bout2)


_NW = 32
_CHUNK = 56
_NGATH = _CHUNK * _K // 128


@functools.lru_cache(maxsize=None)
def _gather_mean_kernel(n_pad, frac0_units=None):
    unit = 2 * _CHUNK
    units_total = n_pad // 16 // unit
    u0 = units_total // 2 if frac0_units is None else frac0_units
    rpw0 = u0 * unit
    rpw1 = (units_total - u0) * unit
    mesh = plsc.VectorSubcoreMesh(core_axis_name="c", subcore_axis_name="s")

    @functools.partial(
        pl.kernel,
        mesh=mesh,
        compiler_params=pltpu.CompilerParams(use_tc_tiling_on_sc=False,
                                             needs_layout_passes=False),
        out_type=jax.ShapeDtypeStruct((n_pad, _GW), jnp.float32),
        scratch_types=[
            pltpu.VMEM((_NGATH, 128), jnp.int32),
            pltpu.VMEM((_NGATH, 128), jnp.int32),
            pltpu.VMEM((_CHUNK * _K, _GW // 2), jnp.int32),
            pltpu.VMEM((_CHUNK * _K, _GW // 2), jnp.int32),
            pltpu.VMEM((_CHUNK, _GW), jnp.float32),
            pltpu.VMEM((_CHUNK, _GW), jnp.float32),
            pltpu.SemaphoreType.DMA,
            pltpu.SemaphoreType.DMA,
            pltpu.SemaphoreType.DMA,
            pltpu.SemaphoreType.DMA,
            pltpu.SemaphoreType.DMA,
            pltpu.SemaphoreType.DMA,
        ],
    )
    def k(idx_hbm, tbl_hbm, out_hbm, idx0, idx1, rows0, rows1, acc0, acc1,
          si0, si1, sr0, sr1, so0, so1):
        idx_b = (idx0, idx1)
        rows_b = (rows0, rows1)
        acc_b = (acc0, acc1)
        si_b = (si0, si1)
        sr_b = (sr0, sr1)
        so_b = (so0, so1)
        c = lax.axis_index("c")
        s = lax.axis_index("s")
        rpw = jnp.where(c == 0, rpw0, rpw1)
        nchunks = rpw // _CHUNK
        base = jnp.where(c == 0, s * rpw0, 16 * rpw0 + s * rpw1)
        ibase = base // 8

        def issue_gathers(b):
            for j in range(_NGATH):
                pltpu.async_copy(tbl_hbm.at[idx_b[b].at[j]],
                                 rows_b[b].at[pl.ds(j * 128, 128)], sr_b[b])

        def wait_gathers(b):
            for j in range(_NGATH):
                pltpu.make_async_copy(tbl_hbm.at[idx_b[b].at[j]],
                                      rows_b[b].at[pl.ds(j * 128, 128)],
                                      sr_b[b]).wait()

        def issue_idx(g, b):
            pltpu.async_copy(
                idx_hbm.at[pl.ds(ibase + g * (_CHUNK // 8), _CHUNK // 8)],
                idx_b[b], si_b[b])

        def wait_idx(g, b):
            pltpu.make_async_copy(
                idx_hbm.at[pl.ds(ibase + g * (_CHUNK // 8), _CHUNK // 8)],
                idx_b[b], si_b[b]).wait()

        def out_copy_args(g, b):
            return (acc_b[b], out_hbm.at[pl.ds(base + g * _CHUNK, _CHUNK)],
                    so_b[b])

        def reduce(b):
            rows = rows_b[b]
            acc = acc_b[b]
            himask = jnp.full((_LANES,), 0xFFFF0000, dtype=jnp.uint32)

            def unpack2(w):
                wu = plsc.bitcast(w, jnp.uint32)
                lo = plsc.bitcast(wu << 16, jnp.float32)
                hi = plsc.bitcast(wu & himask, jnp.float32)
                return lo, hi

            def row_body(r, carry):
                r0 = r * _K
                for cc in range(_GW // 2 // _LANES):
                    slo, shi = unpack2(rows[r0, pl.ds(_LANES * cc, _LANES)])
                    for kk in range(1, _K):
                        lo, hi = unpack2(rows[r0 + kk,
                                              pl.ds(_LANES * cc, _LANES)])
                        slo = slo + lo
                        shi = shi + hi
                    acc[r, pl.ds(_LANES * cc, _LANES)] = slo * (1.0 / _K)
                    acc[r, pl.ds(_GW // 2 + _LANES * cc, _LANES)] = (
                        shi * (1.0 / _K))
                return carry

            lax.fori_loop(0, _CHUNK, row_body, 0)

        def process(g, b):
            wait_gathers(b)
            @pl.when(g + 2 < nchunks)
            def _():
                issue_idx(g + 2, b)
            @pl.when(g + 1 < nchunks)
            def _():
                wait_idx(g + 1, 1 - b)
                issue_gathers(1 - b)
            @pl.when(g >= 2)
            def _():
                pltpu.make_async_copy(*out_copy_args(g - 2, b)).wait()
            reduce(b)
            pltpu.async_copy(*out_copy_args(g, b))

        pltpu.sync_copy(idx_hbm.at[pl.ds(ibase, _CHUNK // 8)], idx0)
        issue_gathers(0)
        issue_idx(1, 1)

        def pair(i, carry):
            process(2 * i, 0)
            process(2 * i + 1, 1)
            return carry

        lax.fori_loop(0, nchunks // 2, pair, 0)
        pltpu.make_async_copy(*out_copy_args(nchunks - 2, 0)).wait()
        pltpu.make_async_copy(*out_copy_args(nchunks - 1, 1)).wait()

    return k


_FRAC0_UNITS = 17


def _gather_mean(idx_flat, tbl, n_pad):
    idx2 = idx_flat.reshape(-1, 128)
    units_total = n_pad // 16 // (2 * _CHUNK)
    u0 = _FRAC0_UNITS if n_pad == 50176 else units_total // 2
    return _gather_mean_kernel(n_pad, u0)(idx2, tbl)


def kernel(coords, cond, knn_idx, B0, B1, B2, W0, b0, Wt, bt, Wg, bg, Wb, bb,
           Wgp, bgp, Wn, Ws, bgc, Wback, Wout, bout):
    n = coords.shape[0]
    align = 2 * _NW * _CHUNK
    if n % align:
        n_pad = ((n + align - 1) // align) * align
    else:
        n_pad = n
    blk = 2 * align if n_pad % (2 * align) == 0 else n_pad
    num = B0.shape[1]

    BB = jnp.concatenate([B0, B1, B2], axis=1)
    sin_rows, cos_rows = [], []
    for i in range(3):
        off = 3 + 2 * num * i
        sin_rows.append(W0[off:off + num])
        cos_rows.append(W0[off + num:off + 2 * num])
    W0c = W0[:3]
    W0s = jnp.concatenate(sin_rows, axis=0)
    W0k = jnp.concatenate(cos_rows, axis=0)

    coordsT = coords.T
    condT = cond.T
    idx_flat = knn_idx.reshape(-1)
    if n_pad != n:
        coordsT = jnp.pad(coordsT, ((0, 0), (0, n_pad - n)))
        condT = jnp.pad(condT, ((0, 0), (0, n_pad - n)))
        idx_flat = jnp.pad(idx_flat, (0, (n_pad - n) * _K))

    bf = jnp.bfloat16
    h, hgp = _trunk(coordsT, condT, BB, W0c, W0s, W0k, b0.reshape(1, -1),
                    Wt.astype(bf), bt, Wg.astype(bf), bg[:, None, :],
                    Wb.astype(bf), bb[:, None, :],
                    Wgp.astype(bf), bgp.reshape(1, -1), blk)

    Wnlo = Wn[:, :_GW // 2, :].astype(bf)
    Wnhi = Wn[:, _GW // 2:, :].astype(bf)
    Wslo = Ws[:, :_GW // 2, :].astype(bf)
    Wshi = Ws[:, _GW // 2:, :].astype(bf)
    for l in range(_GLAYERS - 1):
        agg = _gather_mean(idx_flat, hgp, n_pad)
        hgp = _gc(agg, hgp, Wnlo[l], Wnhi[l], Wslo[l], Wshi[l],
                  bgc[l].reshape(1, -1), blk)
    agg = _gather_mean(idx_flat, hgp, n_pad)

    Wbo = _dot(Wback, Wout)
    uT = _final(h, agg, hgp, Wnlo[_GLAYERS - 1], Wnhi[_GLAYERS - 1],
                Wslo[_GLAYERS - 1], Wshi[_GLAYERS - 1],
                bgc[_GLAYERS - 1].reshape(1, -1),
                Wout, Wbo, bout.reshape(-1, 1), blk)
    return uT[:, :n].T


# speedup vs baseline: 1.0245x; 1.0245x over previous
import functools

import jax
import jax.numpy as jnp
from jax import lax
from jax.experimental import pallas as pl
from jax.experimental.pallas import tpu as pltpu
from jax.experimental.pallas import tpu_sc as plsc

_DEPTH = 7
_RES_SKIPS = (3, 6)
_WIDTH = 128
_GW = 64
_GLAYERS = 4
_K = 16
_LANES = 16
_SCALE = 0.01


def _silu(x):
    return x * jax.nn.sigmoid(x)


def _pack_half(x):
    b = x.astype(jnp.bfloat16)
    lo = lax.bitcast_convert_type(b[:, :_GW // 2], jnp.uint16).astype(jnp.uint32)
    hi = lax.bitcast_convert_type(b[:, _GW // 2:], jnp.uint16).astype(jnp.uint32)
    return lax.bitcast_convert_type(lo | (hi << 16), jnp.int32)


def _unpack_half(w):
    wu = lax.bitcast_convert_type(w, jnp.uint32)
    lo = lax.bitcast_convert_type((wu & 0xFFFF).astype(jnp.uint16),
                                  jnp.bfloat16)
    hi = lax.bitcast_convert_type((wu >> 16).astype(jnp.uint16), jnp.bfloat16)
    return lo, hi


def _dot(a, b):
    return lax.dot_general(a, b, (((1,), (0,)), ((), ())),
                           preferred_element_type=jnp.float32)


def _full_spec(shape):
    zeros = (0,) * len(shape)
    return pl.BlockSpec(shape, lambda i, _z=zeros: _z)


def _row_spec(blk, cols):
    return pl.BlockSpec((blk, cols), lambda i: (i, 0))


def _dot_t(at, b):
    return lax.dot_general(at, b, (((0,), (0,)), ((), ())),
                           preferred_element_type=jnp.float32)


def _trunk_body(coordsT_ref, condT_ref, BB_ref, W0c_ref, W0s_ref, W0k_ref,
                b0_ref, Wt_ref, bt_ref, Wg_ref, bg_ref, Wb_ref, bb_ref,
                Wgp_ref, bgp_ref, h_ref, hg_ref):
    xT = coordsT_ref[...]
    condT = condT_ref[...].astype(jnp.bfloat16)
    PT = _dot_t(BB_ref[...], xT)
    S = jnp.sin(PT)
    C = jnp.cos(PT)
    h = (_dot_t(xT, W0c_ref[...]) + _dot_t(S, W0s_ref[...])
         + _dot_t(C, W0k_ref[...]) + b0_ref[...])

    def film(v, l):
        gamma = _dot_t(condT, Wg_ref[l]) + bg_ref[l]
        beta = _dot_t(condT, Wb_ref[l]) + bb_ref[l]
        return v * (1.0 + gamma) + beta

    h = _silu(film(h, 0))
    h_skip = h
    for l in range(_DEPTH - 1):
        h = _silu(film(_dot(h.astype(jnp.bfloat16), Wt_ref[l]) + bt_ref[l],
                       l + 1))
        if (l + 1) in _RES_SKIPS:
            h = h + h_skip
    h_ref[...] = h
    hg_ref[...] = _pack_half(_silu(_dot(h.astype(jnp.bfloat16), Wgp_ref[...])
                                   + bgp_ref[...]))


def _col_spec(rows, blk):
    return pl.BlockSpec((rows, blk), lambda i: (0, i))


def _trunk(coordsT, condT, BB, W0c, W0s, W0k, b0, Wt, bt, Wg, bg, Wb, bb,
           Wgp, bgp, blk):
    n = coordsT.shape[1]
    grid = n // blk
    args = (coordsT, condT, BB, W0c, W0s, W0k, b0, Wt, bt, Wg, bg, Wb, bb,
            Wgp, bgp)
    in_specs = [_col_spec(3, blk), _col_spec(condT.shape[0], blk)]
    in_specs += [_full_spec(a.shape) for a in args[2:]]
    return pl.pallas_call(
        _trunk_body,
        grid=(grid,),
        in_specs=in_specs,
        out_specs=[_row_spec(blk, _WIDTH), _row_spec(blk, _GW // 2)],
        out_shape=[jax.ShapeDtypeStruct((n, _WIDTH), jnp.float32),
                   jax.ShapeDtypeStruct((n, _GW // 2), jnp.int32)],
    )(*args)


def _gc_body(agg_ref, hgp_ref, Wn_ref, Wslo_ref, Wshi_ref, b_ref, out_ref):
    agg = agg_ref[...].astype(jnp.bfloat16)
    lo, hi = _unpack_half(hgp_ref[...])
    out_ref[...] = _pack_half(_silu(_dot(agg, Wn_ref[...])
                                    + _dot(lo, Wslo_ref[...])
                                    + _dot(hi, Wshi_ref[...]) + b_ref[...]))


def _gc(agg, hgp, Wn_l, Wslo_l, Wshi_l, b_l, blk):
    n = hgp.shape[0]
    grid = n // blk
    return pl.pallas_call(
        _gc_body,
        grid=(grid,),
        in_specs=[_row_spec(blk, _GW), _row_spec(blk, _GW // 2),
                  _full_spec(Wn_l.shape), _full_spec(Wslo_l.shape),
                  _full_spec(Wshi_l.shape), _full_spec(b_l.shape)],
        out_specs=_row_spec(blk, _GW // 2),
        out_shape=jax.ShapeDtypeStruct((n, _GW // 2), jnp.int32),
    )(agg, hgp, Wn_l, Wslo_l, Wshi_l, b_l)


def _final_body(h_ref, agg_ref, hgp_ref, Wn_ref, Wslo_ref, Wshi_ref, b_ref,
                Wout_ref, Wbo_ref, bout_ref, u_ref):
    lo, hi = _unpack_half(hgp_ref[...])
    hg4 = _silu(_dot(agg_ref[...].astype(jnp.bfloat16), Wn_ref[...])
                + _dot(lo, Wslo_ref[...]) + _dot(hi, Wshi_ref[...])
                + b_ref[...])
    contract_last = (((0,), (1,)), ((), ()))
    uT = (lax.dot_general(Wout_ref[...], h_ref[...], contract_last,
                          preferred_element_type=jnp.float32)
          + lax.dot_general(Wbo_ref[...], hg4, contract_last,
                            preferred_element_type=jnp.float32)
          + bout_ref[...]) * _SCALE
    u_ref[...] = uT


def _final(h, agg, hgp, Wn_l, Wslo_l, Wshi_l, b_l, Wout, Wbo, bout2, blk):
    n = h.shape[0]
    grid = n // blk
    out_dim = Wout.shape[1]
    return pl.pallas_call(
        _final_body,
        grid=(grid,),
        in_specs=[_row_spec(blk, _WIDTH), _row_spec(blk, _GW),
                  _row_spec(blk, _GW // 2), _full_spec(Wn_l.shape),
                  _full_spec(Wslo_l.shape), _full_spec(Wshi_l.shape),
                  _full_spec(b_l.shape),
                  _full_spec(Wout.shape), _full_spec(Wbo.shape),
                  _full_spec(bout2.shape)],
        out_specs=_col_spec(out_dim, blk),
        out_shape=jax.ShapeDtypeStruct((out_dim, n), jnp.float32),
    )(h, agg, hgp, Wn_l, Wslo_l, Wshi_l, b_l, Wout, Wbo, bout2)


_NW = 32
_CHUNK = 56
_NGATH = _CHUNK * _K // 128


@functools.lru_cache(maxsize=None)
def _gather_mean_kernel(n_pad, frac0_units=None):
    unit = 2 * _CHUNK
    units_total = n_pad // 16 // unit
    u0 = units_total // 2 if frac0_units is None else frac0_units
    rpw0 = u0 * unit
    rpw1 = (units_total - u0) * unit
    mesh = plsc.VectorSubcoreMesh(core_axis_name="c", subcore_axis_name="s")

    @functools.partial(
        pl.kernel,
        mesh=mesh,
        compiler_params=pltpu.CompilerParams(use_tc_tiling_on_sc=False,
                                             needs_layout_passes=False),
        out_type=jax.ShapeDtypeStruct((n_pad, _GW), jnp.float32),
        scratch_types=[
            pltpu.VMEM((_NGATH, 128), jnp.int32),
            pltpu.VMEM((_NGATH, 128), jnp.int32),
            pltpu.VMEM((_CHUNK * _K, _GW // 2), jnp.int32),
            pltpu.VMEM((_CHUNK * _K, _GW // 2), jnp.int32),
            pltpu.VMEM((_CHUNK, _GW), jnp.float32),
            pltpu.VMEM((_CHUNK, _GW), jnp.float32),
            pltpu.SemaphoreType.DMA,
            pltpu.SemaphoreType.DMA,
            pltpu.SemaphoreType.DMA,
            pltpu.SemaphoreType.DMA,
            pltpu.SemaphoreType.DMA,
            pltpu.SemaphoreType.DMA,
        ],
    )
    def k(idx_hbm, tbl_hbm, out_hbm, idx0, idx1, rows0, rows1, acc0, acc1,
          si0, si1, sr0, sr1, so0, so1):
        idx_b = (idx0, idx1)
        rows_b = (rows0, rows1)
        acc_b = (acc0, acc1)
        si_b = (si0, si1)
        sr_b = (sr0, sr1)
        so_b = (so0, so1)
        c = lax.axis_index("c")
        s = lax.axis_index("s")
        rpw = jnp.where(c == 0, rpw0, rpw1)
        nchunks = rpw // _CHUNK
        base = jnp.where(c == 0, s * rpw0, 16 * rpw0 + s * rpw1)
        ibase = base // 8

        def issue_gathers(b):
            for j in range(_NGATH):
                pltpu.async_copy(tbl_hbm.at[idx_b[b].at[j]],
                                 rows_b[b].at[pl.ds(j * 128, 128)], sr_b[b])

        def wait_gathers(b):
            for j in range(_NGATH):
                pltpu.make_async_copy(tbl_hbm.at[idx_b[b].at[j]],
                                      rows_b[b].at[pl.ds(j * 128, 128)],
                                      sr_b[b]).wait()

        def issue_idx(g, b):
            pltpu.async_copy(
                idx_hbm.at[pl.ds(ibase + g * (_CHUNK // 8), _CHUNK // 8)],
                idx_b[b], si_b[b])

        def wait_idx(g, b):
            pltpu.make_async_copy(
                idx_hbm.at[pl.ds(ibase + g * (_CHUNK // 8), _CHUNK // 8)],
                idx_b[b], si_b[b]).wait()

        def out_copy_args(g, b):
            return (acc_b[b], out_hbm.at[pl.ds(base + g * _CHUNK, _CHUNK)],
                    so_b[b])

        def reduce(b):
            rows = rows_b[b]
            acc = acc_b[b]
            himask = jnp.full((_LANES,), 0xFFFF0000, dtype=jnp.uint32)

            def unpack2(w):
                wu = plsc.bitcast(w, jnp.uint32)
                lo = plsc.bitcast(wu << 16, jnp.float32)
                hi = plsc.bitcast(wu & himask, jnp.float32)
                return lo, hi

            def row_body(r, carry):
                r0 = r * _K
                for cc in range(_GW // 2 // _LANES):
                    slo, shi = unpack2(rows[r0, pl.ds(_LANES * cc, _LANES)])
                    for kk in range(1, _K):
                        lo, hi = unpack2(rows[r0 + kk,
                                              pl.ds(_LANES * cc, _LANES)])
                        slo = slo + lo
                        shi = shi + hi
                    acc[r, pl.ds(_LANES * cc, _LANES)] = slo * (1.0 / _K)
                    acc[r, pl.ds(_GW // 2 + _LANES * cc, _LANES)] = (
                        shi * (1.0 / _K))
                return carry

            lax.fori_loop(0, _CHUNK, row_body, 0)

        def process(g, b):
            wait_gathers(b)
            @pl.when(g + 2 < nchunks)
            def _():
                issue_idx(g + 2, b)
            @pl.when(g + 1 < nchunks)
            def _():
                wait_idx(g + 1, 1 - b)
                issue_gathers(1 - b)
            @pl.when(g >= 2)
            def _():
                pltpu.make_async_copy(*out_copy_args(g - 2, b)).wait()
            reduce(b)
            pltpu.async_copy(*out_copy_args(g, b))

        pltpu.sync_copy(idx_hbm.at[pl.ds(ibase, _CHUNK // 8)], idx0)
        issue_gathers(0)
        issue_idx(1, 1)

        def pair(i, carry):
            process(2 * i, 0)
            process(2 * i + 1, 1)
            return carry

        lax.fori_loop(0, nchunks // 2, pair, 0)
        pltpu.make_async_copy(*out_copy_args(nchunks - 2, 0)).wait()
        pltpu.make_async_copy(*out_copy_args(nchunks - 1, 1)).wait()

    return k


_FRAC0_UNITS = 17


def _gather_mean(idx_flat, tbl, n_pad):
    idx2 = idx_flat.reshape(-1, 128)
    units_total = n_pad // 16 // (2 * _CHUNK)
    u0 = _FRAC0_UNITS if n_pad == 50176 else units_total // 2
    return _gather_mean_kernel(n_pad, u0)(idx2, tbl)


def kernel(coords, cond, knn_idx, B0, B1, B2, W0, b0, Wt, bt, Wg, bg, Wb, bb,
           Wgp, bgp, Wn, Ws, bgc, Wback, Wout, bout):
    n = coords.shape[0]
    align = 2 * _NW * _CHUNK
    if n % align:
        n_pad = ((n + align - 1) // align) * align
    else:
        n_pad = n
    blk = 2 * align if n_pad % (2 * align) == 0 else n_pad
    num = B0.shape[1]

    BB = jnp.concatenate([B0, B1, B2], axis=1)
    sin_rows, cos_rows = [], []
    for i in range(3):
        off = 3 + 2 * num * i
        sin_rows.append(W0[off:off + num])
        cos_rows.append(W0[off + num:off + 2 * num])
    W0c = W0[:3]
    W0s = jnp.concatenate(sin_rows, axis=0)
    W0k = jnp.concatenate(cos_rows, axis=0)

    coordsT = coords.T
    condT = cond.T
    idx_flat = knn_idx.reshape(-1)
    if n_pad != n:
        coordsT = jnp.pad(coordsT, ((0, 0), (0, n_pad - n)))
        condT = jnp.pad(condT, ((0, 0), (0, n_pad - n)))
        idx_flat = jnp.pad(idx_flat, (0, (n_pad - n) * _K))

    bf = jnp.bfloat16
    h, hgp = _trunk(coordsT, condT, BB, W0c, W0s, W0k, b0.reshape(1, -1),
                    Wt.astype(bf), bt, Wg.astype(bf), bg[:, None, :],
                    Wb.astype(bf), bb[:, None, :],
                    Wgp.astype(bf), bgp.reshape(1, -1), blk)

    Wn_b = Wn.astype(bf)
    Wslo = Ws[:, :_GW // 2, :].astype(bf)
    Wshi = Ws[:, _GW // 2:, :].astype(bf)
    for l in range(_GLAYERS - 1):
        agg = _gather_mean(idx_flat, hgp, n_pad)
        hgp = _gc(agg, hgp, Wn_b[l], Wslo[l], Wshi[l],
                  bgc[l].reshape(1, -1), blk)
    agg = _gather_mean(idx_flat, hgp, n_pad)

    Wbo = _dot(Wback, Wout)
    uT = _final(h, agg, hgp, Wn_b[_GLAYERS - 1], Wslo[_GLAYERS - 1],
                Wshi[_GLAYERS - 1], bgc[_GLAYERS - 1].reshape(1, -1),
                Wout, Wbo, bout.reshape(-1, 1), blk)
    return uT[:, :n].T
